# Initial kernel scaffold; baseline (speedup 1.0000x reference)
import jax
import jax.numpy as jnp
from jax.experimental import pallas as pl


def kernel(class_logits, box_regression, handside_logits, dxdymag_regression, contact_regression, proposals):
    raise NotImplementedError("write your pallas kernel here")


# speedup vs baseline: 9.2968x; 9.2968x over previous
import math

import jax
import jax.numpy as jnp
from jax.experimental import pallas as pl
from jax.experimental.pallas import tpu as pltpu

_N = 20000
_NC = 3
_C = _NC - 1
_LANES = 128
_ROWS = 160
_PAD = _ROWS * _LANES
_SROWS = _C * _ROWS
_BLOCKS = _SROWS // 8
_DET = 100
_SCORE_THRESH = 0.05
_NMS_THRESH = 0.5
_CLIP = math.log(1000.0 / 16.0)
_IMG_H, _IMG_W = 800.0, 1066.0
_NEG = float("-inf")
_BIG = 2**30


def _nms_body(in_ref, o_boxes, o_scores, o_labels, o_sides, o_dxdy, o_cont,
              s_ref, bm_ref, bo_ref, bp_ref, kx1, ky1, kx2, ky2):
    f32 = jnp.float32
    lane128 = jax.lax.broadcasted_iota(jnp.int32, (1, _LANES), 1)

    x1 = in_ref[0]
    y1 = in_ref[1]
    x2 = in_ref[2]
    y2 = in_ref[3]
    w = x2 - x1
    h = y2 - y1
    cx = x1 + 0.5 * w
    cy = y1 + 0.5 * h
    l0 = in_ref[16]
    l1 = in_ref[17]
    l2 = in_ref[18]
    mx = jnp.maximum(jnp.maximum(l0, l1), l2)
    e0 = jnp.exp(l0 - mx)
    e1 = jnp.exp(l1 - mx)
    e2 = jnp.exp(l2 - mx)
    den = e0 + e1 + e2
    ri = jax.lax.broadcasted_iota(jnp.int32, (_ROWS, _LANES), 0)
    ci = jax.lax.broadcasted_iota(jnp.int32, (_ROWS, _LANES), 1)
    valid = (ri * _LANES + ci) < _N
    es = (e1, e2)
    for c in range(_C):
        base = 4 + 4 * (c + 1)
        dx = in_ref[base] / 10.0
        dy = in_ref[base + 1] / 10.0
        dw = jnp.minimum(in_ref[base + 2] / 5.0, _CLIP)
        dh = jnp.minimum(in_ref[base + 3] / 5.0, _CLIP)
        px = dx * w + cx
        py = dy * h + cy
        pw = jnp.exp(dw) * w
        ph = jnp.exp(dh) * h
        bx1 = jnp.clip(px - 0.5 * pw, 0.0, _IMG_W)
        by1 = jnp.clip(py - 0.5 * ph, 0.0, _IMG_H)
        bx2 = jnp.clip(px + 0.5 * pw, 0.0, _IMG_W)
        by2 = jnp.clip(py + 0.5 * ph, 0.0, _IMG_H)
        off = f32(4000.0 * (c + 1))
        sl = pl.ds(c * _ROWS, _ROWS)
        bp_ref[0, sl, :] = bx1
        bp_ref[1, sl, :] = by1
        bp_ref[2, sl, :] = bx2
        bp_ref[3, sl, :] = by2
        bo_ref[0, sl, :] = bx1 + off
        bo_ref[1, sl, :] = by1 + off
        bo_ref[2, sl, :] = bx2 + off
        bo_ref[3, sl, :] = by2 + off
        sc = es[c] / den
        s_ref[sl, :] = jnp.where((sc > _SCORE_THRESH) & valid, sc, _NEG)

    bm_ref[...] = jnp.max(s_ref[...].reshape(_BLOCKS, 8, _LANES), axis=1)

    o_boxes[...] = jnp.zeros((_DET, 4), f32)
    o_scores[...] = jnp.zeros((_DET, 1), f32)
    o_labels[...] = jnp.zeros((_DET, 1), jnp.int32)
    o_sides[...] = jnp.zeros((_DET, 1), f32)
    o_dxdy[...] = jnp.zeros((_DET, 3), f32)
    o_cont[...] = jnp.zeros((_DET, 5), f32)
    empty = jnp.full((1, _LANES), -1e30, f32)
    kx1[...] = empty
    ky1[...] = empty
    kx2[...] = empty
    ky2[...] = empty

    def _pick(ref3, plane, r, c):
        row = ref3[plane, pl.ds(r, 1), :]
        return jnp.sum(jnp.where(lane128 == c, row, 0.0))

    def cond(carry):
        return carry[1] > 0

    def body(carry):
        k, _ = carry
        bm = bm_ref[...]
        m = jnp.max(bm)
        alive = m > _NEG
        bi = jax.lax.broadcasted_iota(jnp.int32, (_BLOCKS, _LANES), 0)
        li = jax.lax.broadcasted_iota(jnp.int32, (_BLOCKS, _LANES), 1)
        pos = jnp.min(jnp.where(bm == m, bi * _LANES + li, _BIG))
        b = pos // _LANES
        c = pos - b * _LANES
        blk = s_ref[pl.ds(b * 8, 8), :]
        r8 = jax.lax.broadcasted_iota(jnp.int32, (8, _LANES), 0)
        l8 = jax.lax.broadcasted_iota(jnp.int32, (8, _LANES), 1)
        rr = jnp.min(jnp.where((blk == m) & (l8 == c), r8, _BIG))
        r = b * 8 + rr

        cx1 = _pick(bo_ref, 0, r, c)
        cy1 = _pick(bo_ref, 1, r, c)
        cx2 = _pick(bo_ref, 2, r, c)
        cy2 = _pick(bo_ref, 3, r, c)
        a1 = (cx2 - cx1) * (cy2 - cy1)
        kx1v = kx1[...]
        ky1v = ky1[...]
        kx2v = kx2[...]
        ky2v = ky2[...]
        a2 = (kx2v - kx1v) * (ky2v - ky1v)
        ww = jnp.maximum(jnp.minimum(cx2, kx2v) - jnp.maximum(cx1, kx1v), 0.0)
        hh = jnp.maximum(jnp.minimum(cy2, ky2v) - jnp.maximum(cy1, ky1v), 0.0)
        inter = ww * hh
        iou = inter / (a1 + a2 - inter + 1e-9)
        sup = jnp.max(jnp.where(iou > _NMS_THRESH, 1.0, 0.0)) > 0.0
        keep_now = alive & jnp.logical_not(sup)

        @pl.when(alive)
        def _():
            row = s_ref[pl.ds(r, 1), :]
            s_ref[pl.ds(r, 1), :] = jnp.where(lane128 == c, _NEG, row)
            blk2 = s_ref[pl.ds(b * 8, 8), :]
            bm_ref[pl.ds(b, 1), :] = jnp.max(blk2, axis=0, keepdims=True)

        @pl.when(keep_now)
        def _():
            kx1[...] = jnp.where(lane128 == k, cx1, kx1[...])
            ky1[...] = jnp.where(lane128 == k, cy1, ky1[...])
            kx2[...] = jnp.where(lane128 == k, cx2, kx2[...])
            ky2[...] = jnp.where(lane128 == k, cy2, ky2[...])
            cls = r // _ROWS
            pr = r - cls * _ROWS
            px1 = _pick(bp_ref, 0, r, c)
            py1 = _pick(bp_ref, 1, r, c)
            px2 = _pick(bp_ref, 2, r, c)
            py2 = _pick(bp_ref, 3, r, c)
            l4 = jax.lax.broadcasted_iota(jnp.int32, (1, 4), 1)
            row4 = jnp.where(l4 == 0, px1,
                             jnp.where(l4 == 1, py1,
                                       jnp.where(l4 == 2, px2, py2)))
            o_boxes[pl.ds(k, 1), :] = row4
            o_scores[pl.ds(k, 1), :] = jnp.broadcast_to(m, (1, 1))
            o_labels[pl.ds(k, 1), :] = jnp.broadcast_to(cls + 1, (1, 1))
            side_logit = _pick(in_ref, 19, pr, c)
            side = jnp.where(side_logit > 0.0, 1.0, 0.0)
            o_sides[pl.ds(k, 1), :] = jnp.broadcast_to(side, (1, 1))
            d0 = _pick(in_ref, 20, pr, c)
            d1 = _pick(in_ref, 21, pr, c)
            d2 = _pick(in_ref, 22, pr, c)
            l3 = jax.lax.broadcasted_iota(jnp.int32, (1, 3), 1)
            o_dxdy[pl.ds(k, 1), :] = jnp.where(
                l3 == 0, d0, jnp.where(l3 == 1, d1, d2))
            c0 = _pick(in_ref, 23, pr, c)
            c1 = _pick(in_ref, 24, pr, c)
            c2 = _pick(in_ref, 25, pr, c)
            c3 = _pick(in_ref, 26, pr, c)
            c4 = _pick(in_ref, 27, pr, c)
            l5 = jax.lax.broadcasted_iota(jnp.int32, (1, 5), 1)
            o_cont[pl.ds(k, 1), :] = jnp.where(
                l5 == 0, c0,
                jnp.where(l5 == 1, c1,
                          jnp.where(l5 == 2, c2,
                                    jnp.where(l5 == 3, c3, c4))))

        k2 = k + keep_now.astype(jnp.int32)
        go = (alive & (k2 < _DET)).astype(jnp.int32)
        return (k2, go)

    jax.lax.while_loop(cond, body, (jnp.int32(0), jnp.int32(1)))


def kernel(class_logits, box_regression, handside_logits, dxdymag_regression,
           contact_regression, proposals):
    planes = jnp.concatenate(
        [proposals, box_regression, class_logits, handside_logits,
         dxdymag_regression, contact_regression], axis=1)
    planes = jnp.pad(planes.T, ((0, 0), (0, _PAD - _N)))
    planes = planes.reshape(28, _ROWS, _LANES)

    f32 = jnp.float32
    out_shape = [
        jax.ShapeDtypeStruct((_DET, 4), f32),
        jax.ShapeDtypeStruct((_DET, 1), f32),
        jax.ShapeDtypeStruct((_DET, 1), jnp.int32),
        jax.ShapeDtypeStruct((_DET, 1), f32),
        jax.ShapeDtypeStruct((_DET, 3), f32),
        jax.ShapeDtypeStruct((_DET, 5), f32),
    ]
    scratch_shapes = [
        pltpu.VMEM((_SROWS, _LANES), f32),
        pltpu.VMEM((_BLOCKS, _LANES), f32),
        pltpu.VMEM((4, _SROWS, _LANES), f32),
        pltpu.VMEM((4, _SROWS, _LANES), f32),
        pltpu.VMEM((1, _LANES), f32),
        pltpu.VMEM((1, _LANES), f32),
        pltpu.VMEM((1, _LANES), f32),
        pltpu.VMEM((1, _LANES), f32),
    ]
    b, s, lab, sd, dd, ct = pl.pallas_call(
        _nms_body, out_shape=out_shape, scratch_shapes=scratch_shapes)(planes)
    return b, s[:, 0], lab[:, 0], sd[:, 0], dd, ct


# speedup vs baseline: 11.6842x; 1.2568x over previous
import math

import jax
import jax.numpy as jnp
from jax.experimental import pallas as pl
from jax.experimental.pallas import tpu as pltpu

_N = 20000
_NC = 3
_C = _NC - 1
_LANES = 128
_ROWS = 160
_PAD = _ROWS * _LANES
_SROWS = _C * _ROWS
_BLOCKS = _SROWS // 8
_DET = 100
_SCORE_THRESH = 0.05
_NMS_THRESH = 0.5
_CLIP = math.log(1000.0 / 16.0)
_IMG_H, _IMG_W = 800.0, 1066.0
_NEG = float("-inf")
_BIG = 2**30


def _nms_body(in_ref, o_boxes, o_scores, o_labels, o_sides, o_dxdy, o_cont,
              s_ref, bo_ref, bp_ref, kx1, ky1, kx2, ky2):
    f32 = jnp.float32
    lane128 = jax.lax.broadcasted_iota(jnp.int32, (1, _LANES), 1)

    x1 = in_ref[0]
    y1 = in_ref[1]
    x2 = in_ref[2]
    y2 = in_ref[3]
    w = x2 - x1
    h = y2 - y1
    cx = x1 + 0.5 * w
    cy = y1 + 0.5 * h
    l0 = in_ref[16]
    l1 = in_ref[17]
    l2 = in_ref[18]
    mx = jnp.maximum(jnp.maximum(l0, l1), l2)
    e0 = jnp.exp(l0 - mx)
    e1 = jnp.exp(l1 - mx)
    e2 = jnp.exp(l2 - mx)
    den = e0 + e1 + e2
    ri = jax.lax.broadcasted_iota(jnp.int32, (_ROWS, _LANES), 0)
    ci = jax.lax.broadcasted_iota(jnp.int32, (_ROWS, _LANES), 1)
    valid = (ri * _LANES + ci) < _N
    es = (e1, e2)
    for c in range(_C):
        base = 4 + 4 * (c + 1)
        dx = in_ref[base] / 10.0
        dy = in_ref[base + 1] / 10.0
        dw = jnp.minimum(in_ref[base + 2] / 5.0, _CLIP)
        dh = jnp.minimum(in_ref[base + 3] / 5.0, _CLIP)
        px = dx * w + cx
        py = dy * h + cy
        pw = jnp.exp(dw) * w
        ph = jnp.exp(dh) * h
        bx1 = jnp.clip(px - 0.5 * pw, 0.0, _IMG_W)
        by1 = jnp.clip(py - 0.5 * ph, 0.0, _IMG_H)
        bx2 = jnp.clip(px + 0.5 * pw, 0.0, _IMG_W)
        by2 = jnp.clip(py + 0.5 * ph, 0.0, _IMG_H)
        off = f32(4000.0 * (c + 1))
        sl = pl.ds(c * _ROWS, _ROWS)
        bp_ref[0, sl, :] = bx1
        bp_ref[1, sl, :] = by1
        bp_ref[2, sl, :] = bx2
        bp_ref[3, sl, :] = by2
        bo_ref[0, sl, :] = bx1 + off
        bo_ref[1, sl, :] = by1 + off
        bo_ref[2, sl, :] = bx2 + off
        bo_ref[3, sl, :] = by2 + off
        sc = es[c] / den
        s_ref[sl, :] = jnp.where((sc > _SCORE_THRESH) & valid, sc, _NEG)

    o_boxes[...] = jnp.zeros((_DET, 4), f32)
    o_scores[...] = jnp.zeros((_DET, 1), f32)
    o_labels[...] = jnp.zeros((_DET, 1), jnp.int32)
    o_sides[...] = jnp.zeros((_DET, 1), f32)
    o_dxdy[...] = jnp.zeros((_DET, 3), f32)
    o_cont[...] = jnp.zeros((_DET, 5), f32)
    empty = jnp.full((1, _LANES), -1e30, f32)
    kx1[...] = empty
    ky1[...] = empty
    kx2[...] = empty
    ky2[...] = empty

    def _pick(ref3, plane, r, c):
        row = ref3[plane, pl.ds(r, 1), :]
        return jnp.sum(jnp.where(lane128 == c, row, 0.0))

    sri = jax.lax.broadcasted_iota(jnp.int32, (_SROWS, _LANES), 0)
    sci = jax.lax.broadcasted_iota(jnp.int32, (_SROWS, _LANES), 1)
    sflat = sri * _LANES + sci

    def _argmax():
        sv = s_ref[...]
        mk = jnp.max(sv, axis=(0, 1), keepdims=True)
        m = mk[0, 0]
        pos = jnp.min(jnp.where(sv == jnp.broadcast_to(mk, sv.shape),
                                sflat, _BIG))
        return m, pos

    def cond(carry):
        return carry[3] > 0

    def body(carry):
        pos, m, k, _ = carry
        r = pos // _LANES
        c = pos - r * _LANES

        row = s_ref[pl.ds(r, 1), :]
        s_ref[pl.ds(r, 1), :] = jnp.where(lane128 == c, _NEG, row)
        m_next, pos_next = _argmax()

        cx1 = _pick(bo_ref, 0, r, c)
        cy1 = _pick(bo_ref, 1, r, c)
        cx2 = _pick(bo_ref, 2, r, c)
        cy2 = _pick(bo_ref, 3, r, c)
        a1 = (cx2 - cx1) * (cy2 - cy1)
        kx1v = kx1[...]
        ky1v = ky1[...]
        kx2v = kx2[...]
        ky2v = ky2[...]
        a2 = (kx2v - kx1v) * (ky2v - ky1v)
        ww = jnp.maximum(jnp.minimum(cx2, kx2v) - jnp.maximum(cx1, kx1v), 0.0)
        hh = jnp.maximum(jnp.minimum(cy2, ky2v) - jnp.maximum(cy1, ky1v), 0.0)
        inter = ww * hh
        iou = inter / (a1 + a2 - inter + 1e-9)
        sup = jnp.max(jnp.where(iou > _NMS_THRESH, 1.0, 0.0)) > 0.0
        keep_now = jnp.logical_not(sup)

        @pl.when(keep_now)
        def _():
            kx1[...] = jnp.where(lane128 == k, cx1, kx1[...])
            ky1[...] = jnp.where(lane128 == k, cy1, ky1[...])
            kx2[...] = jnp.where(lane128 == k, cx2, kx2[...])
            ky2[...] = jnp.where(lane128 == k, cy2, ky2[...])
            cls = r // _ROWS
            pr = r - cls * _ROWS
            px1 = _pick(bp_ref, 0, r, c)
            py1 = _pick(bp_ref, 1, r, c)
            px2 = _pick(bp_ref, 2, r, c)
            py2 = _pick(bp_ref, 3, r, c)
            l4 = jax.lax.broadcasted_iota(jnp.int32, (1, 4), 1)
            row4 = jnp.where(l4 == 0, px1,
                             jnp.where(l4 == 1, py1,
                                       jnp.where(l4 == 2, px2, py2)))
            o_boxes[pl.ds(k, 1), :] = row4
            o_scores[pl.ds(k, 1), :] = jnp.broadcast_to(m, (1, 1))
            o_labels[pl.ds(k, 1), :] = jnp.broadcast_to(cls + 1, (1, 1))
            side_logit = _pick(in_ref, 19, pr, c)
            side = jnp.where(side_logit > 0.0, 1.0, 0.0)
            o_sides[pl.ds(k, 1), :] = jnp.broadcast_to(side, (1, 1))
            d0 = _pick(in_ref, 20, pr, c)
            d1 = _pick(in_ref, 21, pr, c)
            d2 = _pick(in_ref, 22, pr, c)
            l3 = jax.lax.broadcasted_iota(jnp.int32, (1, 3), 1)
            o_dxdy[pl.ds(k, 1), :] = jnp.where(
                l3 == 0, d0, jnp.where(l3 == 1, d1, d2))
            c0 = _pick(in_ref, 23, pr, c)
            c1 = _pick(in_ref, 24, pr, c)
            c2 = _pick(in_ref, 25, pr, c)
            c3 = _pick(in_ref, 26, pr, c)
            c4 = _pick(in_ref, 27, pr, c)
            l5 = jax.lax.broadcasted_iota(jnp.int32, (1, 5), 1)
            o_cont[pl.ds(k, 1), :] = jnp.where(
                l5 == 0, c0,
                jnp.where(l5 == 1, c1,
                          jnp.where(l5 == 2, c2,
                                    jnp.where(l5 == 3, c3, c4))))

        k2 = k + keep_now.astype(jnp.int32)
        go = ((m_next > _NEG) & (k2 < _DET)).astype(jnp.int32)
        return (pos_next, m_next, k2, go)

    m0, pos0 = _argmax()
    jax.lax.while_loop(
        cond, body,
        (pos0, m0, jnp.int32(0), (m0 > _NEG).astype(jnp.int32)))


def kernel(class_logits, box_regression, handside_logits, dxdymag_regression,
           contact_regression, proposals):
    planes = jnp.concatenate(
        [proposals, box_regression, class_logits, handside_logits,
         dxdymag_regression, contact_regression], axis=1)
    planes = jnp.pad(planes.T, ((0, 0), (0, _PAD - _N)))
    planes = planes.reshape(28, _ROWS, _LANES)

    f32 = jnp.float32
    out_shape = [
        jax.ShapeDtypeStruct((_DET, 4), f32),
        jax.ShapeDtypeStruct((_DET, 1), f32),
        jax.ShapeDtypeStruct((_DET, 1), jnp.int32),
        jax.ShapeDtypeStruct((_DET, 1), f32),
        jax.ShapeDtypeStruct((_DET, 3), f32),
        jax.ShapeDtypeStruct((_DET, 5), f32),
    ]
    scratch_shapes = [
        pltpu.VMEM((_SROWS, _LANES), f32),
        pltpu.VMEM((4, _SROWS, _LANES), f32),
        pltpu.VMEM((4, _SROWS, _LANES), f32),
        pltpu.VMEM((1, _LANES), f32),
        pltpu.VMEM((1, _LANES), f32),
        pltpu.VMEM((1, _LANES), f32),
        pltpu.VMEM((1, _LANES), f32),
    ]
    b, s, lab, sd, dd, ct = pl.pallas_call(
        _nms_body, out_shape=out_shape, scratch_shapes=scratch_shapes)(planes)
    return b, s[:, 0], lab[:, 0], sd[:, 0], dd, ct


# speedup vs baseline: 16.0306x; 1.3720x over previous
import math

import jax
import jax.numpy as jnp
from jax.experimental import pallas as pl
from jax.experimental.pallas import tpu as pltpu

_N = 20000
_NC = 3
_C = _NC - 1
_LANES = 128
_ROWS = 160
_PAD = _ROWS * _LANES
_SROWS = _C * _ROWS
_BLOCKS = _SROWS // 8
_DET = 100
_SCORE_THRESH = 0.05
_NMS_THRESH = 0.5
_CLIP = math.log(1000.0 / 16.0)
_IMG_H, _IMG_W = 800.0, 1066.0
_NEG = float("-inf")
_BIG = 2**30


def _nms_body(in_ref, o_boxes, o_scores, o_labels, o_sides, o_dxdy, o_cont,
              s_ref, bo_ref, bp_ref, kx1, ky1, kx2, ky2):
    f32 = jnp.float32
    lane128 = jax.lax.broadcasted_iota(jnp.int32, (1, _LANES), 1)

    x1 = in_ref[0]
    y1 = in_ref[1]
    x2 = in_ref[2]
    y2 = in_ref[3]
    w = x2 - x1
    h = y2 - y1
    cx = x1 + 0.5 * w
    cy = y1 + 0.5 * h
    l0 = in_ref[16]
    l1 = in_ref[17]
    l2 = in_ref[18]
    mx = jnp.maximum(jnp.maximum(l0, l1), l2)
    e0 = jnp.exp(l0 - mx)
    e1 = jnp.exp(l1 - mx)
    e2 = jnp.exp(l2 - mx)
    den = e0 + e1 + e2
    ri = jax.lax.broadcasted_iota(jnp.int32, (_ROWS, _LANES), 0)
    ci = jax.lax.broadcasted_iota(jnp.int32, (_ROWS, _LANES), 1)
    valid = (ri * _LANES + ci) < _N
    es = (e1, e2)
    for c in range(_C):
        base = 4 + 4 * (c + 1)
        dx = in_ref[base] / 10.0
        dy = in_ref[base + 1] / 10.0
        dw = jnp.minimum(in_ref[base + 2] / 5.0, _CLIP)
        dh = jnp.minimum(in_ref[base + 3] / 5.0, _CLIP)
        px = dx * w + cx
        py = dy * h + cy
        pw = jnp.exp(dw) * w
        ph = jnp.exp(dh) * h
        bx1 = jnp.clip(px - 0.5 * pw, 0.0, _IMG_W)
        by1 = jnp.clip(py - 0.5 * ph, 0.0, _IMG_H)
        bx2 = jnp.clip(px + 0.5 * pw, 0.0, _IMG_W)
        by2 = jnp.clip(py + 0.5 * ph, 0.0, _IMG_H)
        off = f32(4000.0 * (c + 1))
        sl = pl.ds(c * _ROWS, _ROWS)
        bp_ref[0, sl, :] = bx1
        bp_ref[1, sl, :] = by1
        bp_ref[2, sl, :] = bx2
        bp_ref[3, sl, :] = by2
        bo_ref[0, sl, :] = bx1 + off
        bo_ref[1, sl, :] = by1 + off
        bo_ref[2, sl, :] = bx2 + off
        bo_ref[3, sl, :] = by2 + off
        sc = es[c] / den
        s_ref[sl, :] = jnp.where((sc > _SCORE_THRESH) & valid, sc, _NEG)

    o_boxes[...] = jnp.zeros((_DET, 4), f32)
    o_scores[...] = jnp.zeros((_DET, 1), f32)
    o_labels[...] = jnp.zeros((_DET, 1), jnp.int32)
    o_sides[...] = jnp.zeros((_DET, 1), f32)
    o_dxdy[...] = jnp.zeros((_DET, 3), f32)
    o_cont[...] = jnp.zeros((_DET, 5), f32)
    empty = jnp.full((1, _LANES), -1e30, f32)
    kx1[...] = empty
    ky1[...] = empty
    kx2[...] = empty
    ky2[...] = empty

    def _pickv(ref3, plane, r, c):
        row = ref3[plane, pl.ds(r, 1), :]
        return jnp.sum(jnp.where(lane128 == c, row, 0.0), axis=1,
                       keepdims=True)

    sri = jax.lax.broadcasted_iota(jnp.int32, (_SROWS, _LANES), 0)
    sci = jax.lax.broadcasted_iota(jnp.int32, (_SROWS, _LANES), 1)
    sflat = sri * _LANES + sci

    def _next_cand():
        sv = s_ref[...]
        mk = jnp.max(sv, axis=(0, 1), keepdims=True)
        enc = jnp.min(jnp.where((sv > _NEG) &
                                (sv == jnp.broadcast_to(mk, sv.shape)),
                                sflat, _BIG))
        return mk, enc

    def cond(carry):
        return carry[3] > 0

    def body(carry):
        pos, mv, k, _ = carry
        r = pos // _LANES
        c = pos - r * _LANES

        row = s_ref[pl.ds(r, 1), :]
        s_ref[pl.ds(r, 1), :] = jnp.where(lane128 == c, _NEG, row)
        mv_next, pos_next = _next_cand()

        cx1 = _pickv(bo_ref, 0, r, c)
        cy1 = _pickv(bo_ref, 1, r, c)
        cx2 = _pickv(bo_ref, 2, r, c)
        cy2 = _pickv(bo_ref, 3, r, c)
        a1 = (cx2 - cx1) * (cy2 - cy1)
        kx1v = kx1[...]
        ky1v = ky1[...]
        kx2v = kx2[...]
        ky2v = ky2[...]
        a2 = (kx2v - kx1v) * (ky2v - ky1v)
        ww = jnp.maximum(jnp.minimum(cx2, kx2v) - jnp.maximum(cx1, kx1v), 0.0)
        hh = jnp.maximum(jnp.minimum(cy2, ky2v) - jnp.maximum(cy1, ky1v), 0.0)
        inter = ww * hh
        iou = inter / (a1 + a2 - inter + 1e-9)
        keep = jnp.max(jnp.where(iou > _NMS_THRESH, 1.0, 0.0)) == 0.0

        ins = (lane128 == k) & keep
        kx1[...] = jnp.where(ins, jnp.broadcast_to(cx1, (1, _LANES)), kx1v)
        ky1[...] = jnp.where(ins, jnp.broadcast_to(cy1, (1, _LANES)), ky1v)
        kx2[...] = jnp.where(ins, jnp.broadcast_to(cx2, (1, _LANES)), kx2v)
        ky2[...] = jnp.where(ins, jnp.broadcast_to(cy2, (1, _LANES)), ky2v)

        cls = r // _ROWS
        pr = r - cls * _ROWS
        px1 = _pickv(bp_ref, 0, r, c)
        py1 = _pickv(bp_ref, 1, r, c)
        px2 = _pickv(bp_ref, 2, r, c)
        py2 = _pickv(bp_ref, 3, r, c)
        l4 = jax.lax.broadcasted_iota(jnp.int32, (1, 4), 1)
        row4 = jnp.where(l4 == 0, px1,
                         jnp.where(l4 == 1, py1,
                                   jnp.where(l4 == 2, px2, py2)))
        o_boxes[pl.ds(k, 1), :] = jnp.where(keep, row4,
                                            o_boxes[pl.ds(k, 1), :])
        o_scores[pl.ds(k, 1), :] = jnp.where(keep, mv,
                                             o_scores[pl.ds(k, 1), :])
        lab = jnp.broadcast_to(cls + 1, (1, 1)).astype(jnp.int32)
        o_labels[pl.ds(k, 1), :] = jnp.where(keep, lab,
                                             o_labels[pl.ds(k, 1), :])
        sidev = _pickv(in_ref, 19, pr, c)
        side = jnp.where(sidev > 0.0, 1.0, 0.0)
        o_sides[pl.ds(k, 1), :] = jnp.where(keep, side,
                                            o_sides[pl.ds(k, 1), :])
        d0 = _pickv(in_ref, 20, pr, c)
        d1 = _pickv(in_ref, 21, pr, c)
        d2 = _pickv(in_ref, 22, pr, c)
        l3 = jax.lax.broadcasted_iota(jnp.int32, (1, 3), 1)
        row3 = jnp.where(l3 == 0, d0, jnp.where(l3 == 1, d1, d2))
        o_dxdy[pl.ds(k, 1), :] = jnp.where(keep, row3,
                                           o_dxdy[pl.ds(k, 1), :])
        c0 = _pickv(in_ref, 23, pr, c)
        c1 = _pickv(in_ref, 24, pr, c)
        c2 = _pickv(in_ref, 25, pr, c)
        c3 = _pickv(in_ref, 26, pr, c)
        c4 = _pickv(in_ref, 27, pr, c)
        l5 = jax.lax.broadcasted_iota(jnp.int32, (1, 5), 1)
        row5 = jnp.where(l5 == 0, c0,
                         jnp.where(l5 == 1, c1,
                                   jnp.where(l5 == 2, c2,
                                             jnp.where(l5 == 3, c3, c4))))
        o_cont[pl.ds(k, 1), :] = jnp.where(keep, row5,
                                           o_cont[pl.ds(k, 1), :])

        k2 = k + keep.astype(jnp.int32)
        go = ((pos_next < _BIG) & (k2 < _DET)).astype(jnp.int32)
        return (pos_next, mv_next, k2, go)

    mv0, pos0 = _next_cand()
    jax.lax.while_loop(
        cond, body,
        (pos0, mv0, jnp.int32(0), (pos0 < _BIG).astype(jnp.int32)))


def kernel(class_logits, box_regression, handside_logits, dxdymag_regression,
           contact_regression, proposals):
    planes = jnp.concatenate(
        [proposals, box_regression, class_logits, handside_logits,
         dxdymag_regression, contact_regression], axis=1)
    planes = jnp.pad(planes.T, ((0, 0), (0, _PAD - _N)))
    planes = planes.reshape(28, _ROWS, _LANES)

    f32 = jnp.float32
    out_shape = [
        jax.ShapeDtypeStruct((_DET, 4), f32),
        jax.ShapeDtypeStruct((_DET, 1), f32),
        jax.ShapeDtypeStruct((_DET, 1), jnp.int32),
        jax.ShapeDtypeStruct((_DET, 1), f32),
        jax.ShapeDtypeStruct((_DET, 3), f32),
        jax.ShapeDtypeStruct((_DET, 5), f32),
    ]
    scratch_shapes = [
        pltpu.VMEM((_SROWS, _LANES), f32),
        pltpu.VMEM((4, _SROWS, _LANES), f32),
        pltpu.VMEM((4, _SROWS, _LANES), f32),
        pltpu.VMEM((1, _LANES), f32),
        pltpu.VMEM((1, _LANES), f32),
        pltpu.VMEM((1, _LANES), f32),
        pltpu.VMEM((1, _LANES), f32),
    ]
    b, s, lab, sd, dd, ct = pl.pallas_call(
        _nms_body, out_shape=out_shape, scratch_shapes=scratch_shapes)(planes)
    return b, s[:, 0], lab[:, 0], sd[:, 0], dd, ct
